# baseline (speedup 1.0000x reference)
import functools

import jax
import jax.numpy as jnp
from jax import lax
from jax.experimental import pallas as pl
from jax.experimental.pallas import tpu as pltpu
from jax.experimental.pallas import tpu_sc as plsc

B = 100000
V = 87
D = 256
C = 80
NC = 2
NS = 16
NW = NC * NS
NCHUNKS = B // C
NLOC = 40
BPAD = NW * NLOC * C

LANES = 16
COLB = D // LANES
GPC = C // LANES
NBUF = 4


TSLICE = V * D // NS


def _body(at_no_hbm, table_hbm, out_hbm, table_sh, table_v, idx_v, rows_v,
          base_sm, sem_i, sem_s):
    c = lax.axis_index("c")
    s = lax.axis_index("s")
    wid = s * NC + c
    chunk0 = 39 * wid + jnp.minimum(wid, 2)
    nloc = jnp.where(wid < 2, NLOC, NLOC - 1)
    aligned0 = (chunk0 // 8) * 8
    off = chunk0 - aligned0

    h_idx = pltpu.async_copy(at_no_hbm.at[pl.ds(aligned0, NLOC + 8)], idx_v, sem_i)
    pltpu.sync_copy(
        table_hbm.at[pl.ds(s * TSLICE, TSLICE)],
        table_v.at[pl.ds(s * TSLICE, TSLICE)],
    )
    pltpu.sync_copy(
        table_v.at[pl.ds(s * TSLICE, TSLICE)],
        table_sh.at[pl.ds(s * TSLICE, TSLICE)],
    )
    plsc.subcore_barrier()
    pltpu.sync_copy(table_sh, table_v)
    h_idx.wait()

    def construct_chunk(j, b):
        for q in range(GPC):
            vec = idx_v[off + j, pl.ds(q * LANES, LANES)] * D
            for r in range(LANES):
                base_sm[q * LANES + r] = vec[r]

        @plsc.parallel_loop(0, C, unroll=16)
        def _(i):
            base = base_sm[i]
            for k in range(COLB):
                rows_v[b, i, pl.ds(k * LANES, LANES)] = table_v[
                    pl.ds(base + k * LANES, LANES)
                ]

    def group(g, carry):
        for b in range(NBUF):
            j = g * NBUF + b
            cid = chunk0 + j

            @pl.when(j < nloc)
            def _():
                @pl.when(g > 0)
                def _():
                    pltpu.make_async_copy(
                        rows_v.at[b], out_hbm.at[pl.ds(0, C)], sem_s.at[b]
                    ).wait()

                construct_chunk(j, b)
                pltpu.async_copy(
                    rows_v.at[b], out_hbm.at[pl.ds(cid * C, C)], sem_s.at[b]
                )
        return carry

    lax.fori_loop(0, NLOC // NBUF, group, 0)

    for b in range(NBUF):
        pltpu.make_async_copy(
            rows_v.at[b], out_hbm.at[pl.ds(0, C)], sem_s.at[b]
        ).wait()


def kernel(at_no, embed_ten):
    at_no_p = jnp.concatenate(
        [at_no, jnp.zeros((BPAD - B,), dtype=at_no.dtype)]
    ).reshape(NW * NLOC, C)
    mesh = plsc.VectorSubcoreMesh(core_axis_name="c", subcore_axis_name="s")
    k = functools.partial(
        pl.kernel,
        mesh=mesh,
        compiler_params=pltpu.CompilerParams(needs_layout_passes=False),
        out_type=jax.ShapeDtypeStruct((B, D), jnp.float32),
        scratch_types=[
            pltpu.VMEM_SHARED((V * D,), jnp.float32),
            pltpu.VMEM((V * D,), jnp.float32),
            pltpu.VMEM((NLOC + 8, C), jnp.int32),
            pltpu.VMEM((NBUF, C, D), jnp.float32),
            pltpu.SMEM((C,), jnp.int32),
            pltpu.SemaphoreType.DMA,
            pltpu.SemaphoreType.DMA((NBUF,)),
        ],
    )(_body)
    return k(at_no_p, embed_ten.reshape(V * D))


# speedup vs baseline: 1.3098x; 1.3098x over previous
import functools

import jax
import jax.numpy as jnp
from jax import lax
from jax.experimental import pallas as pl
from jax.experimental.pallas import tpu as pltpu
from jax.experimental.pallas import tpu_sc as plsc

B = 100000
V = 87
D = 256
C = 80
NC = 2
NS = 16
NW = NC * NS
NCHUNKS = B // C
NLOC = 40
BPAD = NW * NLOC * C

LANES = 16
COLB = D // LANES
GPC = C // LANES
NBUF = 4


TSLICE = V * D // NS


def _body(at_no_hbm, table_hbm, out_hbm, table_sh, table_v, idx_v, rows_v,
          base_sm, sem_i, sem_s):
    c = lax.axis_index("c")
    s = lax.axis_index("s")
    wid = s * NC + c
    chunk0 = 39 * wid + jnp.minimum(wid, 2)
    nloc = jnp.where(wid < 2, NLOC, NLOC - 1)
    aligned0 = (chunk0 // 8) * 8
    off = chunk0 - aligned0

    h_idx = pltpu.async_copy(at_no_hbm.at[pl.ds(aligned0, NLOC + 8)], idx_v, sem_i)
    pltpu.sync_copy(
        table_hbm.at[pl.ds(s * TSLICE, TSLICE)],
        table_v.at[pl.ds(s * TSLICE, TSLICE)],
    )
    pltpu.sync_copy(
        table_v.at[pl.ds(s * TSLICE, TSLICE)],
        table_sh.at[pl.ds(s * TSLICE, TSLICE)],
    )
    plsc.subcore_barrier()
    pltpu.sync_copy(table_sh, table_v)
    h_idx.wait()

    def construct_chunk(j, b):
        for q in range(GPC):
            vec = idx_v[off + j, pl.ds(q * LANES, LANES)] * D
            for r in range(LANES):
                base_sm[q * LANES + r] = vec[r]

        @plsc.parallel_loop(0, C, unroll=8)
        def _(i):
            base = base_sm[i]
            for k in range(COLB):
                rows_v[b, i, pl.ds(k * LANES, LANES)] = table_v[
                    pl.ds(base + k * LANES, LANES)
                ]

    def group(g, carry):
        for b in range(NBUF):
            j = g * NBUF + b
            cid = chunk0 + j

            @pl.when(j < nloc)
            def _():
                @pl.when(g > 0)
                def _():
                    pltpu.make_async_copy(
                        rows_v.at[b], out_hbm.at[pl.ds(0, C)], sem_s.at[b]
                    ).wait()

                construct_chunk(j, b)
                pltpu.async_copy(
                    rows_v.at[b], out_hbm.at[pl.ds(cid * C, C)], sem_s.at[b]
                )
        return carry

    lax.fori_loop(0, NLOC // NBUF, group, 0)

    for b in range(NBUF):
        pltpu.make_async_copy(
            rows_v.at[b], out_hbm.at[pl.ds(0, C)], sem_s.at[b]
        ).wait()


def kernel(at_no, embed_ten):
    at_no_p = jnp.concatenate(
        [at_no, jnp.zeros((BPAD - B,), dtype=at_no.dtype)]
    ).reshape(NW * NLOC, C)
    mesh = plsc.VectorSubcoreMesh(core_axis_name="c", subcore_axis_name="s")
    k = functools.partial(
        pl.kernel,
        mesh=mesh,
        compiler_params=pltpu.CompilerParams(needs_layout_passes=False),
        out_type=jax.ShapeDtypeStruct((B, D), jnp.float32),
        scratch_types=[
            pltpu.VMEM_SHARED((V * D,), jnp.float32),
            pltpu.VMEM((V * D,), jnp.float32),
            pltpu.VMEM((NLOC + 8, C), jnp.int32),
            pltpu.VMEM((NBUF, C, D), jnp.float32),
            pltpu.SMEM((C,), jnp.int32),
            pltpu.SemaphoreType.DMA,
            pltpu.SemaphoreType.DMA((NBUF,)),
        ],
    )(_body)
    return k(at_no_p, embed_ten.reshape(V * D))
